# baseline (speedup 1.0000x reference)
import jax
import jax.numpy as jnp
from jax import lax
from jax.experimental import pallas as pl
from jax.experimental.pallas import tpu as pltpu
from jax.experimental.pallas import tpu_sc as plsc

_NC = 2
_NS = 16
_NW = _NC * _NS
_ROWS = 1_000_000
_CHUNK = 400
_NCHUNKS = _ROWS // _CHUNK
_NFULL = _NCHUNKS // _NW
_EXTRA = _NCHUNKS - _NFULL * _NW
_KMAX = _NFULL + 1


def _sc_copy(in_hbm, out_hbm, buf0, buf1, rs0, rs1, ws0, ws1):
    wid = lax.axis_index("s") * _NC + lax.axis_index("c")
    cnt = _NFULL + jnp.where(wid < _EXTRA, 1, 0)
    base = (wid * _NFULL + jnp.minimum(wid, _EXTRA)) * _CHUNK
    bufs = (buf0, buf1)
    rsem = (rs0, rs1)
    wsem = (ws0, ws1)

    def rd(k):
        b = k % 2
        return pltpu.make_async_copy(
            in_hbm.at[pl.ds(base + k * _CHUNK, _CHUNK), :], bufs[b], rsem[b])

    def wr(k):
        b = k % 2
        return pltpu.make_async_copy(
            bufs[b], out_hbm.at[pl.ds(base + k * _CHUNK, _CHUNK), :], wsem[b])

    def guarded(k, op):
        if k < _NFULL:
            op()
        else:
            pl.when(k < cnt)(op)

    rd(0).start()
    for k in range(_KMAX):
        if k >= 1:
            wr(k - 1).wait()
        guarded(k, rd(k).wait)
        if k + 1 < _KMAX:
            guarded(k + 1, rd(k + 1).start)
        guarded(k, wr(k).start)

    @pl.when(cnt == _KMAX)
    def _():
        wr(_KMAX - 1).wait()


def kernel(table):
    n, d = table.shape
    mesh = plsc.VectorSubcoreMesh(core_axis_name="c", subcore_axis_name="s")
    run = pl.kernel(
        _sc_copy,
        out_type=jax.ShapeDtypeStruct((n, d), table.dtype),
        mesh=mesh,
        scratch_types=[
            pltpu.VMEM((_CHUNK, 32), jnp.float32),
            pltpu.VMEM((_CHUNK, 32), jnp.float32),
            pltpu.SemaphoreType.DMA,
            pltpu.SemaphoreType.DMA,
            pltpu.SemaphoreType.DMA,
            pltpu.SemaphoreType.DMA,
        ],
    )
    return run(table)


# speedup vs baseline: 1.0092x; 1.0092x over previous
import jax
import jax.numpy as jnp
from jax import lax
from jax.experimental import pallas as pl
from jax.experimental.pallas import tpu as pltpu
from jax.experimental.pallas import tpu_sc as plsc

_NC = 2
_NS = 16
_NW = _NC * _NS
_ROWS = 1_000_000
_CHUNK = 400
_NCHUNKS = _ROWS // _CHUNK
_NFULL = _NCHUNKS // _NW
_EXTRA = _NCHUNKS - _NFULL * _NW
_KMAX = _NFULL + 1
_D = 2
_AHEAD = 1


def _sc_copy(in_hbm, out_hbm, *refs):
    bufs = refs[:_D]
    rsem = refs[_D:2 * _D]
    wsem = refs[2 * _D:3 * _D]
    wid = lax.axis_index("s") * _NC + lax.axis_index("c")
    cnt = _NFULL + jnp.where(wid < _EXTRA, 1, 0)
    base = (wid * _NFULL + jnp.minimum(wid, _EXTRA)) * _CHUNK

    def rd(k):
        b = k % _D
        return pltpu.make_async_copy(
            in_hbm.at[pl.ds(base + k * _CHUNK, _CHUNK), :], bufs[b], rsem[b])

    def wr(k):
        b = k % _D
        return pltpu.make_async_copy(
            bufs[b], out_hbm.at[pl.ds(base + k * _CHUNK, _CHUNK), :], wsem[b])

    def guarded(k, op):
        if k < _NFULL:
            op()
        else:
            pl.when(k < cnt)(op)

    for k in range(min(_AHEAD, _KMAX)):
        guarded(k, rd(k).start)
    for k in range(_KMAX):
        if k >= _AHEAD:
            guarded(k - _AHEAD, wr(k - _AHEAD).wait)
        if k + _AHEAD < _KMAX:
            guarded(k + _AHEAD, rd(k + _AHEAD).start)
        guarded(k, rd(k).wait)
        guarded(k, wr(k).start)
    for k in range(max(_KMAX - _AHEAD, 0), _KMAX):
        guarded(k, wr(k).wait)


def kernel(table):
    n, d = table.shape
    mesh = plsc.VectorSubcoreMesh(core_axis_name="c", subcore_axis_name="s")
    run = pl.kernel(
        _sc_copy,
        out_type=jax.ShapeDtypeStruct((n, d), table.dtype),
        mesh=mesh,
        compiler_params=pltpu.CompilerParams(use_tc_tiling_on_sc=True),
        scratch_types=(
            [pltpu.VMEM((_CHUNK, 32), jnp.float32) for _ in range(_D)]
            + [pltpu.SemaphoreType.DMA for _ in range(2 * _D)]
        ),
    )
    return run(table)


# speedup vs baseline: 1.0669x; 1.0572x over previous
import jax
import jax.numpy as jnp
from jax.experimental import pallas as pl
from jax.experimental.pallas import tpu as pltpu


def _copy_block(in_ref, out_ref):
    out_ref[...] = in_ref[...]


def kernel(table):
    n, d = table.shape
    block = 25000
    return pl.pallas_call(
        _copy_block,
        grid=(n // block,),
        in_specs=[pl.BlockSpec((block, d), lambda i: (i, 0))],
        out_specs=pl.BlockSpec((block, d), lambda i: (i, 0)),
        out_shape=jax.ShapeDtypeStruct((n, d), table.dtype),
        compiler_params=pltpu.CompilerParams(
            dimension_semantics=("arbitrary",),
        ),
    )(table)
